# baseline (speedup 1.0000x reference)
import numpy as np

import jax
import jax.numpy as jnp
from jax import lax
from jax.experimental import pallas as pl
from jax.experimental.pallas import tpu as pltpu
from jax.experimental.pallas import tpu_sc as plsc

N_ATOMS = 10000
N_BONDS = 320000
MAX_NB = 32
HID = 128
NLC = 8

NC, NS = 2, 16
NW = NC * NS

BA = 8
NBA = 40
APW = BA * NBA
PA = NW * APW

NCH = 5
CHB = N_BONDS // NCH
BPW = CHB // NW
BB = 80
NBB = BPW // BB

MM_BLK = 2000


def _atom_body(a2b_hbm, ma_hbm, mbond_hbm, out_hbm,
               idx_all, r0a, r0b, r1a, r1b, acc, s0, s1):
    wid = lax.axis_index("s") * NC + lax.axis_index("c")
    abase = wid * APW
    pltpu.sync_copy(a2b_hbm.at[wid], idx_all)
    pltpu.sync_copy(ma_hbm.at[pl.ds(abase, APW)], acc)

    def gath(b, bufa, bufb, sem):
        bc = jnp.minimum(b, NBA - 1)
        pltpu.async_copy(mbond_hbm.at[idx_all.at[2 * bc]], bufa, sem)
        pltpu.async_copy(mbond_hbm.at[idx_all.at[2 * bc + 1]], bufb, sem)

    def waitg(bufa, bufb, sem):
        pltpu.make_async_copy(mbond_hbm.at[idx_all.at[0]], bufa, sem).wait()
        pltpu.make_async_copy(mbond_hbm.at[idx_all.at[0]], bufb, sem).wait()

    def compute(b, bufa, bufb):
        def half(buf, half_idx):
            def atom(i, carry2):
                r0 = i * MAX_NB
                v0 = [buf[r0, pl.ds(16 * c, 16)] for c in range(NLC)]

                def red(j, a):
                    vs = [buf[r0 + j, pl.ds(16 * c, 16)] for c in range(NLC)]
                    s = [a[c] + vs[c] for c in range(NLC)]
                    m = [jnp.maximum(a[NLC + c], vs[c]) for c in range(NLC)]
                    return tuple(s + m)

                a = lax.fori_loop(1, MAX_NB, red, tuple(v0 + v0), unroll=2)
                row = b * BA + half_idx * (BA // 2) + i
                for c in range(NLC):
                    sl = pl.ds(16 * c, 16)
                    acc[row, sl] = acc[row, sl] + a[c] * a[NLC + c]
                return carry2

            lax.fori_loop(0, BA // 2, atom, 0)

        half(bufa, 0)
        half(bufb, 1)

    gath(0, r0a, r0b, s0)
    gath(1, r1a, r1b, s1)

    def pair(t, carry):
        b0 = 2 * t
        waitg(r0a, r0b, s0)
        compute(b0, r0a, r0b)
        gath(b0 + 2, r0a, r0b, s0)
        waitg(r1a, r1b, s1)
        compute(b0 + 1, r1a, r1b)
        gath(b0 + 3, r1a, r1b, s1)
        return carry

    lax.fori_loop(0, NBA // 2, pair, 0)
    waitg(r0a, r0b, s0)
    waitg(r1a, r1b, s1)
    pltpu.sync_copy(acc, out_hbm.at[pl.ds(abase, APW)])


_atom_kernel = pl.kernel(
    _atom_body,
    out_type=jax.ShapeDtypeStruct((PA, HID), jnp.float32),
    mesh=plsc.VectorSubcoreMesh(core_axis_name="c", subcore_axis_name="s"),
    scratch_types=[
        pltpu.VMEM((2 * NBA, 128), jnp.int32),
        pltpu.VMEM((128, HID), jnp.float32),
        pltpu.VMEM((128, HID), jnp.float32),
        pltpu.VMEM((128, HID), jnp.float32),
        pltpu.VMEM((128, HID), jnp.float32),
        pltpu.VMEM((APW, HID), jnp.float32),
        pltpu.SemaphoreType.DMA,
        pltpu.SemaphoreType.DMA,
    ],
)


def _bond_body(b2a_hbm, b2revb_hbm, manew_hbm, mbond_hbm, g_hbm,
               idx_a, idx_r, ra0, rr0, ra1, rr1, ob0, ob1,
               sg0, sg1, so0, so1):
    wid = lax.axis_index("s") * NC + lax.axis_index("c")
    bbase = wid * BPW
    pltpu.sync_copy(b2a_hbm.at[wid], idx_a)
    pltpu.sync_copy(b2revb_hbm.at[wid], idx_r)

    def gath(k, ra, rr, sg):
        pltpu.async_copy(manew_hbm.at[idx_a.at[k]], ra, sg)
        pltpu.async_copy(mbond_hbm.at[idx_r.at[k]], rr, sg)

    def waitg(ra, rr, sg):
        pltpu.make_async_copy(manew_hbm.at[idx_a.at[0]], ra, sg).wait()
        pltpu.make_async_copy(mbond_hbm.at[idx_r.at[0]], rr, sg).wait()

    def waitst(ob, so):
        pltpu.make_async_copy(ob, g_hbm.at[pl.ds(bbase, BB)], so).wait()

    def rne_hi(u):
        return (u + jnp.int32(0x7FFF) + ((u >> 16) & jnp.int32(1))) >> 16

    def comp_st(k, ra, rr, ob, so):
        def row(i, carry2):
            for c2 in range(NLC // 2):
                sl0 = pl.ds(32 * c2, 16)
                sl1 = pl.ds(32 * c2 + 16, 16)
                d0 = ra[i, sl0] - rr[i, sl0]
                d1 = ra[i, sl1] - rr[i, sl1]
                u0 = plsc.bitcast(d0, jnp.int32)
                u1 = plsc.bitcast(d1, jnp.int32)
                lo = rne_hi(u0) & jnp.int32(0xFFFF)
                hi = rne_hi(u1) << 16
                ob[i, pl.ds(16 * c2, 16)] = lo | hi
            return carry2

        lax.fori_loop(0, BB, row, 0)
        pltpu.async_copy(ob, g_hbm.at[pl.ds(bbase + BB * k, BB)], so)

    gath(0, ra0, rr0, sg0)
    gath(1, ra1, rr1, sg1)
    waitg(ra0, rr0, sg0)
    comp_st(0, ra0, rr0, ob0, so0)
    gath(2, ra0, rr0, sg0)
    waitg(ra1, rr1, sg1)
    comp_st(1, ra1, rr1, ob1, so1)
    gath(3, ra1, rr1, sg1)

    def pair(t, carry):
        b = 2 * t + 2
        waitg(ra0, rr0, sg0)
        waitst(ob0, so0)
        comp_st(b, ra0, rr0, ob0, so0)
        gath(jnp.minimum(b + 2, NBB - 1), ra0, rr0, sg0)
        waitg(ra1, rr1, sg1)
        waitst(ob1, so1)
        comp_st(b + 1, ra1, rr1, ob1, so1)
        gath(jnp.minimum(b + 3, NBB - 1), ra1, rr1, sg1)
        return carry

    lax.fori_loop(0, (NBB - 3) // 2, pair, 0)
    waitg(ra0, rr0, sg0)
    waitst(ob0, so0)
    comp_st(NBB - 1, ra0, rr0, ob0, so0)
    waitg(ra1, rr1, sg1)
    waitst(ob0, so0)
    waitst(ob1, so1)


_bond_kernel = pl.kernel(
    _bond_body,
    out_type=jax.ShapeDtypeStruct((CHB, HID // 2), jnp.int32),
    mesh=plsc.VectorSubcoreMesh(core_axis_name="c", subcore_axis_name="s"),
    compiler_params=pltpu.CompilerParams(needs_layout_passes=False),
    scratch_types=[
        pltpu.VMEM((NBB, BB), jnp.int32),
        pltpu.VMEM((NBB, BB), jnp.int32),
        pltpu.VMEM((BB, HID), jnp.float32),
        pltpu.VMEM((BB, HID), jnp.float32),
        pltpu.VMEM((BB, HID), jnp.float32),
        pltpu.VMEM((BB, HID), jnp.float32),
        pltpu.VMEM((BB, HID // 2), jnp.int32),
        pltpu.VMEM((BB, HID // 2), jnp.int32),
        pltpu.SemaphoreType.DMA,
        pltpu.SemaphoreType.DMA,
        pltpu.SemaphoreType.DMA,
        pltpu.SemaphoreType.DMA,
    ],
)


def _mm_compute(g_ref, in_ref, w0_ref, w1_ref, b_ref, o_ref):
    x = g_ref[...]
    d0 = lax.bitcast_convert_type(x << 16, jnp.float32)
    d1 = lax.bitcast_convert_type(x & jnp.int32(-65536), jnp.float32)
    mm = jnp.dot(d0, w0_ref[...], preferred_element_type=jnp.float32)
    mm = mm + jnp.dot(d1, w1_ref[...], preferred_element_type=jnp.float32)
    o_ref[...] = jnp.maximum(in_ref[...] + mm + b_ref[...], 0.0)


def _mm_body(g_ref, in_ref, w0_ref, w1_ref, b_ref, o_ref):
    _mm_compute(g_ref, in_ref, w0_ref, w1_ref, b_ref, o_ref)


def _mm_body_acc(m_ref, g_ref, in_ref, w0_ref, w1_ref, b_ref, o_ref):
    _mm_compute(g_ref, in_ref, w0_ref, w1_ref, b_ref, o_ref)


def _linear_relu_chunk(j, m, g_j, input_bond, w0, w1, b2d):
    grid = CHB // MM_BLK
    gspec = pl.BlockSpec((MM_BLK, HID // 2), lambda i: (i, 0))
    inspec = pl.BlockSpec((MM_BLK, HID), lambda i, j=j: (j * grid + i, 0))
    wspec = pl.BlockSpec((HID // 2, HID), lambda i: (0, 0))
    bspec = pl.BlockSpec((1, HID), lambda i: (0, 0))
    outspec = pl.BlockSpec((MM_BLK, HID), lambda i, j=j: (j * grid + i, 0))
    out_shape = jax.ShapeDtypeStruct((N_BONDS, HID), jnp.float32)
    if j == 0:
        return pl.pallas_call(
            _mm_body,
            grid=(grid,),
            in_specs=[gspec, inspec, wspec, wspec, bspec],
            out_specs=outspec,
            out_shape=out_shape,
        )(g_j, input_bond, w0, w1, b2d)
    mspec = pl.BlockSpec((8, HID), lambda i: (0, 0))
    return pl.pallas_call(
        _mm_body_acc,
        grid=(grid,),
        in_specs=[mspec, gspec, inspec, wspec, wspec, bspec],
        out_specs=outspec,
        out_shape=out_shape,
        input_output_aliases={0: 0},
    )(m, g_j, input_bond, w0, w1, b2d)


def kernel(message_atom, message_bond, a2b, b2a, b2revb, input_bond, W_bond, b_bond):
    a2b = a2b.astype(jnp.int32)
    b2a = b2a.astype(jnp.int32)
    b2revb = b2revb.astype(jnp.int32)

    ma_pad = jnp.pad(message_atom, ((0, PA - N_ATOMS), (0, 0)))
    pad_idx = jnp.arange((PA - N_ATOMS) * MAX_NB, dtype=jnp.int32) % N_BONDS
    a2b_pad = jnp.concatenate([a2b.reshape(-1), pad_idx])
    a2b_pad = a2b_pad.reshape(NW, 2 * NBA, 128)
    b2a_r = b2a.reshape(NCH, NW, NBB, BB)
    b2revb_r = b2revb.reshape(NCH, NW, NBB, BB)

    manew_pad = _atom_kernel(a2b_pad, ma_pad, message_bond)
    jcol = np.arange(HID // 2)
    idx0 = 32 * (jcol // 16) + jcol % 16
    wt = W_bond.T
    w0 = wt[idx0]
    w1 = wt[idx0 + 16]
    b2d = b_bond.reshape(1, HID)
    mb = None
    for j in range(NCH):
        g_j = _bond_kernel(b2a_r[j], b2revb_r[j], manew_pad, message_bond)
        mb = _linear_relu_chunk(j, mb, g_j, input_bond, w0, w1, b2d)
    return (manew_pad[:N_ATOMS], mb)


# speedup vs baseline: 1.0658x; 1.0658x over previous
import numpy as np

import jax
import jax.numpy as jnp
from jax import lax
from jax.experimental import pallas as pl
from jax.experimental.pallas import tpu as pltpu
from jax.experimental.pallas import tpu_sc as plsc

N_ATOMS = 10000
N_BONDS = 320000
MAX_NB = 32
HID = 128
NLC = 8

NC, NS = 2, 16
NW = NC * NS

BA = 8
NBA = 40
APW = BA * NBA
PA = NW * APW

NCH = 5
CHB = N_BONDS // NCH
BPW = CHB // NW
BB = 80
NBB = BPW // BB

MM_BLK = 2000


def _atom_body(a2b_hbm, ma_hbm, mbond_hbm, out_hbm,
               idx_all, r0a, r0b, r1a, r1b, acc, s0, s1):
    wid = lax.axis_index("s") * NC + lax.axis_index("c")
    abase = wid * APW
    pltpu.sync_copy(a2b_hbm.at[wid], idx_all)
    pltpu.sync_copy(ma_hbm.at[pl.ds(abase, APW)], acc)

    def gath(b, bufa, bufb, sem):
        bc = jnp.minimum(b, NBA - 1)
        pltpu.async_copy(mbond_hbm.at[idx_all.at[2 * bc]], bufa, sem)
        pltpu.async_copy(mbond_hbm.at[idx_all.at[2 * bc + 1]], bufb, sem)

    def waitg(bufa, bufb, sem):
        pltpu.make_async_copy(mbond_hbm.at[idx_all.at[0]], bufa, sem).wait()
        pltpu.make_async_copy(mbond_hbm.at[idx_all.at[0]], bufb, sem).wait()

    def compute(b, bufa, bufb):
        def half(buf, half_idx):
            def atom(i, carry2):
                r0 = i * MAX_NB
                v0 = [buf[r0, pl.ds(16 * c, 16)] for c in range(NLC)]

                def red(j, a):
                    vs = [buf[r0 + j, pl.ds(16 * c, 16)] for c in range(NLC)]
                    s = [a[c] + vs[c] for c in range(NLC)]
                    m = [jnp.maximum(a[NLC + c], vs[c]) for c in range(NLC)]
                    return tuple(s + m)

                a = lax.fori_loop(1, MAX_NB, red, tuple(v0 + v0), unroll=2)
                row = b * BA + half_idx * (BA // 2) + i
                for c in range(NLC):
                    sl = pl.ds(16 * c, 16)
                    acc[row, sl] = acc[row, sl] + a[c] * a[NLC + c]
                return carry2

            lax.fori_loop(0, BA // 2, atom, 0)

        half(bufa, 0)
        half(bufb, 1)

    gath(0, r0a, r0b, s0)
    gath(1, r1a, r1b, s1)

    def pair(t, carry):
        b0 = 2 * t
        waitg(r0a, r0b, s0)
        compute(b0, r0a, r0b)
        gath(b0 + 2, r0a, r0b, s0)
        waitg(r1a, r1b, s1)
        compute(b0 + 1, r1a, r1b)
        gath(b0 + 3, r1a, r1b, s1)
        return carry

    lax.fori_loop(0, NBA // 2, pair, 0)
    waitg(r0a, r0b, s0)
    waitg(r1a, r1b, s1)
    pltpu.sync_copy(acc, out_hbm.at[pl.ds(abase, APW)])


_atom_kernel = pl.kernel(
    _atom_body,
    out_type=jax.ShapeDtypeStruct((PA, HID), jnp.float32),
    mesh=plsc.VectorSubcoreMesh(core_axis_name="c", subcore_axis_name="s"),
    scratch_types=[
        pltpu.VMEM((2 * NBA, 128), jnp.int32),
        pltpu.VMEM((128, HID), jnp.float32),
        pltpu.VMEM((128, HID), jnp.float32),
        pltpu.VMEM((128, HID), jnp.float32),
        pltpu.VMEM((128, HID), jnp.float32),
        pltpu.VMEM((APW, HID), jnp.float32),
        pltpu.SemaphoreType.DMA,
        pltpu.SemaphoreType.DMA,
    ],
)


def _bond_body(b2a_hbm, b2revb_hbm, manew_hbm, mbond_hbm, g_hbm,
               idx_a, idx_r, ra0, rr0, ra1, rr1, ob0, ob1,
               sg0, sg1, so0, so1):
    wid = lax.axis_index("s") * NC + lax.axis_index("c")
    bbase2 = wid * (BPW // 2)
    pltpu.sync_copy(b2a_hbm.at[wid], idx_a)
    pltpu.sync_copy(b2revb_hbm.at[wid], idx_r)

    def gath(k, ra, rr, sg):
        pltpu.async_copy(manew_hbm.at[idx_a.at[k]], ra, sg)
        pltpu.async_copy(mbond_hbm.at[idx_r.at[k]], rr, sg)

    def waitg(ra, rr, sg):
        pltpu.make_async_copy(manew_hbm.at[idx_a.at[0]], ra, sg).wait()
        pltpu.make_async_copy(mbond_hbm.at[idx_r.at[0]], rr, sg).wait()

    def waitst(ob, so):
        pltpu.make_async_copy(ob, g_hbm.at[pl.ds(bbase2, BB // 2)], so).wait()

    def rne_hi(u):
        return (u + jnp.int32(0x7FFF) + ((u >> 16) & jnp.int32(1))) >> 16

    def pack_pair(ra, rr, i, c2):
        sl0 = pl.ds(32 * c2, 16)
        sl1 = pl.ds(32 * c2 + 16, 16)
        d0 = ra[i, sl0] - rr[i, sl0]
        d1 = ra[i, sl1] - rr[i, sl1]
        u0 = plsc.bitcast(d0, jnp.int32)
        u1 = plsc.bitcast(d1, jnp.int32)
        return (rne_hi(u0) & jnp.int32(0xFFFF)) | (rne_hi(u1) << 16)

    def comp_st(k, ra, rr, ob, so):
        def row(i, carry2):
            for c2 in range(NLC // 2):
                ob[i, pl.ds(16 * c2, 16)] = pack_pair(ra, rr, i, c2)
                ob[i, pl.ds(64 + 16 * c2, 16)] = pack_pair(ra, rr, BB // 2 + i, c2)
            return carry2

        lax.fori_loop(0, BB // 2, row, 0)
        pltpu.async_copy(ob, g_hbm.at[pl.ds(bbase2 + (BB // 2) * k, BB // 2)], so)

    gath(0, ra0, rr0, sg0)
    gath(1, ra1, rr1, sg1)
    waitg(ra0, rr0, sg0)
    comp_st(0, ra0, rr0, ob0, so0)
    gath(2, ra0, rr0, sg0)
    waitg(ra1, rr1, sg1)
    comp_st(1, ra1, rr1, ob1, so1)
    gath(3, ra1, rr1, sg1)

    def pair(t, carry):
        b = 2 * t + 2
        waitg(ra0, rr0, sg0)
        waitst(ob0, so0)
        comp_st(b, ra0, rr0, ob0, so0)
        gath(jnp.minimum(b + 2, NBB - 1), ra0, rr0, sg0)
        waitg(ra1, rr1, sg1)
        waitst(ob1, so1)
        comp_st(b + 1, ra1, rr1, ob1, so1)
        gath(jnp.minimum(b + 3, NBB - 1), ra1, rr1, sg1)
        return carry

    lax.fori_loop(0, (NBB - 3) // 2, pair, 0)
    waitg(ra0, rr0, sg0)
    waitst(ob0, so0)
    comp_st(NBB - 1, ra0, rr0, ob0, so0)
    waitg(ra1, rr1, sg1)
    waitst(ob0, so0)
    waitst(ob1, so1)


_bond_kernel = pl.kernel(
    _bond_body,
    out_type=jax.ShapeDtypeStruct((CHB // 2, HID), jnp.int32),
    mesh=plsc.VectorSubcoreMesh(core_axis_name="c", subcore_axis_name="s"),
    compiler_params=pltpu.CompilerParams(needs_layout_passes=False),
    scratch_types=[
        pltpu.VMEM((NBB, BB), jnp.int32),
        pltpu.VMEM((NBB, BB), jnp.int32),
        pltpu.VMEM((BB, HID), jnp.float32),
        pltpu.VMEM((BB, HID), jnp.float32),
        pltpu.VMEM((BB, HID), jnp.float32),
        pltpu.VMEM((BB, HID), jnp.float32),
        pltpu.VMEM((BB // 2, HID), jnp.int32),
        pltpu.VMEM((BB // 2, HID), jnp.int32),
        pltpu.SemaphoreType.DMA,
        pltpu.SemaphoreType.DMA,
        pltpu.SemaphoreType.DMA,
        pltpu.SemaphoreType.DMA,
    ],
)


def _mm_compute(g_ref, in_ref, w0_ref, w1_ref, b_ref, o_ref):
    x = g_ref[...]
    half = MM_BLK // 2
    for h in range(2):
        xh = x[:, 64 * h:64 * h + 64]
        d0 = lax.bitcast_convert_type(xh << 16, jnp.float32)
        d1 = lax.bitcast_convert_type(xh & jnp.int32(-65536), jnp.float32)
        mm = jnp.dot(d0, w0_ref[...], preferred_element_type=jnp.float32)
        mm = mm + jnp.dot(d1, w1_ref[...], preferred_element_type=jnp.float32)
        sl = pl.ds(half * h, half)
        o_ref[sl, :] = jnp.maximum(in_ref[sl, :] + mm + b_ref[...], 0.0)


def _mm_body(g_ref, in_ref, w0_ref, w1_ref, b_ref, o_ref):
    _mm_compute(g_ref, in_ref, w0_ref, w1_ref, b_ref, o_ref)


def _mm_body_acc(m_ref, g_ref, in_ref, w0_ref, w1_ref, b_ref, o_ref):
    _mm_compute(g_ref, in_ref, w0_ref, w1_ref, b_ref, o_ref)


def _linear_relu_chunk(j, m, g_j, input_bond, w0, w1, b2d):
    grid = CHB // MM_BLK
    gspec = pl.BlockSpec((MM_BLK // 2, HID), lambda i: (i, 0))
    inspec = pl.BlockSpec((MM_BLK, HID), lambda i, j=j: (j * grid + i, 0))
    wspec = pl.BlockSpec((HID // 2, HID), lambda i: (0, 0))
    bspec = pl.BlockSpec((1, HID), lambda i: (0, 0))
    outspec = pl.BlockSpec((MM_BLK, HID), lambda i, j=j: (j * grid + i, 0))
    out_shape = jax.ShapeDtypeStruct((N_BONDS, HID), jnp.float32)
    if j == 0:
        return pl.pallas_call(
            _mm_body,
            grid=(grid,),
            in_specs=[gspec, inspec, wspec, wspec, bspec],
            out_specs=outspec,
            out_shape=out_shape,
        )(g_j, input_bond, w0, w1, b2d)
    mspec = pl.BlockSpec((8, HID), lambda i: (0, 0))
    return pl.pallas_call(
        _mm_body_acc,
        grid=(grid,),
        in_specs=[mspec, gspec, inspec, wspec, wspec, bspec],
        out_specs=outspec,
        out_shape=out_shape,
        input_output_aliases={0: 0},
    )(m, g_j, input_bond, w0, w1, b2d)


def kernel(message_atom, message_bond, a2b, b2a, b2revb, input_bond, W_bond, b_bond):
    a2b = a2b.astype(jnp.int32)
    b2a = b2a.astype(jnp.int32)
    b2revb = b2revb.astype(jnp.int32)

    ma_pad = jnp.pad(message_atom, ((0, PA - N_ATOMS), (0, 0)))
    pad_idx = jnp.arange((PA - N_ATOMS) * MAX_NB, dtype=jnp.int32) % N_BONDS
    a2b_pad = jnp.concatenate([a2b.reshape(-1), pad_idx])
    a2b_pad = a2b_pad.reshape(NW, 2 * NBA, 128)
    def _pair_layout(ix):
        ix = ix.reshape(NCH, NW, 2, NBB, BB // 2)
        return ix.transpose(0, 1, 3, 2, 4).reshape(NCH, NW, NBB, BB)

    b2a_r = _pair_layout(b2a)
    b2revb_r = _pair_layout(b2revb)

    manew_pad = _atom_kernel(a2b_pad, ma_pad, message_bond)
    jcol = np.arange(HID // 2)
    idx0 = 32 * (jcol // 16) + jcol % 16
    wt = W_bond.T
    w0 = wt[idx0]
    w1 = wt[idx0 + 16]
    b2d = b_bond.reshape(1, HID)
    mb = None
    for j in range(NCH):
        g_j = _bond_kernel(b2a_r[j], b2revb_r[j], manew_pad, message_bond)
        mb = _linear_relu_chunk(j, mb, g_j, input_bond, w0, w1, b2d)
    return (manew_pad[:N_ATOMS], mb)
